# baseline (speedup 1.0000x reference)
import functools

import jax
import jax.numpy as jnp
from jax import lax
from jax.experimental import pallas as pl
from jax.experimental.pallas import tpu as pltpu
from jax.experimental.pallas import tpu_sc as plsc

_N_BINS = 15
_BLOCK_ROWS = 2048


def _stats_kernel(x_ref, w_ref, b_ref, conf_ref):
    x = x_ref[...]
    logits = jnp.dot(x, w_ref[...], preferred_element_type=jnp.float32) + b_ref[...]
    m = jnp.max(logits, axis=1, keepdims=True)
    e = jnp.exp(logits - m)
    s = jnp.sum(e, axis=1, keepdims=True)
    conf_ref[...] = 1.0 / s


def _scale_kernel(x_ref, w_ref, b_ref, est_ref, out_ref):
    x = x_ref[...]
    logits = jnp.dot(x, w_ref[...], preferred_element_type=jnp.float32) + b_ref[...]
    m = jnp.max(logits, axis=1, keepdims=True)
    e = jnp.exp(logits - m)
    s = jnp.sum(e, axis=1, keepdims=True)
    est = est_ref[...]
    t = (1.0 - est) / (s - 1.0)
    out_ref[...] = jnp.where(logits == m, est, e * t)


def _sc_bin_lookup(conf, hist16):
    batch = conf.shape[0]
    n_workers = 32
    bpw = batch // n_workers
    mesh = plsc.VectorSubcoreMesh(core_axis_name="c", subcore_axis_name="s")

    @functools.partial(
        pl.kernel,
        mesh=mesh,
        out_type=jax.ShapeDtypeStruct((batch,), jnp.float32),
        scratch_types=[
            pltpu.VMEM((bpw,), jnp.float32),
            pltpu.VMEM((16,), jnp.float32),
            pltpu.VMEM((bpw,), jnp.float32),
        ],
        compiler_params=pltpu.CompilerParams(needs_layout_passes=False),
    )
    def k(conf_hbm, hist_hbm, est_hbm, conf_v, hist_v, est_v):
        wid = lax.axis_index("s") * 2 + lax.axis_index("c")
        base = wid * bpw
        pltpu.sync_copy(conf_hbm.at[pl.ds(base, bpw)], conf_v)
        pltpu.sync_copy(hist_hbm, hist_v)
        for i in range(bpw // 16):
            c = conf_v[pl.ds(i * 16, 16)]
            y = c * float(_N_BINS)
            ti = y.astype(jnp.int32)
            idx = ti - jnp.where(ti.astype(jnp.float32) == y, 1, 0)
            idx = jnp.minimum(jnp.maximum(idx, 0), _N_BINS - 1)
            hv = plsc.load_gather(hist_v, [idx])
            est_v[pl.ds(i * 16, 16)] = jnp.where(hv == -1.0, c, hv)
        pltpu.sync_copy(est_v, est_hbm.at[pl.ds(base, bpw)])

    return k(conf, hist16)


def kernel(x, W, b, histogram):
    batch, d_in = x.shape
    n_classes = W.shape[1]
    b2 = b.reshape(1, n_classes)
    hist16 = jnp.zeros((16,), jnp.float32).at[:_N_BINS].set(histogram)
    grid = (batch // _BLOCK_ROWS,)
    conf2d = pl.pallas_call(
        _stats_kernel,
        grid=grid,
        in_specs=[
            pl.BlockSpec((_BLOCK_ROWS, d_in), lambda i: (i, 0)),
            pl.BlockSpec((d_in, n_classes), lambda i: (0, 0)),
            pl.BlockSpec((1, n_classes), lambda i: (0, 0)),
        ],
        out_specs=pl.BlockSpec((_BLOCK_ROWS, 1), lambda i: (i, 0)),
        out_shape=jax.ShapeDtypeStruct((batch, 1), jnp.float32),
    )(x, W, b2)
    est = _sc_bin_lookup(conf2d.reshape(batch), hist16)
    return pl.pallas_call(
        _scale_kernel,
        grid=grid,
        in_specs=[
            pl.BlockSpec((_BLOCK_ROWS, d_in), lambda i: (i, 0)),
            pl.BlockSpec((d_in, n_classes), lambda i: (0, 0)),
            pl.BlockSpec((1, n_classes), lambda i: (0, 0)),
            pl.BlockSpec((_BLOCK_ROWS, 1), lambda i: (i, 0)),
        ],
        out_specs=pl.BlockSpec((_BLOCK_ROWS, n_classes), lambda i: (i, 0)),
        out_shape=jax.ShapeDtypeStruct((batch, n_classes), jnp.float32),
    )(x, W, b2, est.reshape(batch, 1))


# speedup vs baseline: 1.5483x; 1.5483x over previous
import jax
import jax.numpy as jnp
from jax.experimental import pallas as pl

_N_BINS = 15
_BLOCK_ROWS = 1024


def _calib_kernel(x_ref, w_ref, b_ref, hist_ref, out_ref):
    x = x_ref[...]
    logits = jnp.dot(x, w_ref[...], preferred_element_type=jnp.float32) + b_ref[...]
    m = jnp.max(logits, axis=1, keepdims=True)
    e = jnp.exp(logits - m)
    s = jnp.sum(e, axis=1, keepdims=True)
    conf = 1.0 / s
    rows = logits.shape[0]
    onehot = logits == m
    idx = jnp.clip(jnp.ceil(conf * _N_BINS).astype(jnp.int32) - 1, 0, _N_BINS - 1)
    hist = hist_ref[...]
    bins = jax.lax.broadcasted_iota(jnp.int32, (rows, 16), 1)
    hist_val = jnp.sum(jnp.where(bins == idx, hist, 0.0), axis=1, keepdims=True)
    est = jnp.where(hist_val == -1.0, conf, hist_val)
    t = (1.0 - est) / (s - 1.0)
    out_ref[...] = jnp.where(onehot, est, e * t)


def kernel(x, W, b, histogram):
    batch, d_in = x.shape
    n_classes = W.shape[1]
    hist_p = jnp.zeros((1, 16), jnp.float32).at[0, :_N_BINS].set(histogram)
    b2 = b.reshape(1, n_classes)
    return pl.pallas_call(
        _calib_kernel,
        grid=(batch // _BLOCK_ROWS,),
        in_specs=[
            pl.BlockSpec((_BLOCK_ROWS, d_in), lambda i: (i, 0)),
            pl.BlockSpec((d_in, n_classes), lambda i: (0, 0)),
            pl.BlockSpec((1, n_classes), lambda i: (0, 0)),
            pl.BlockSpec((1, 16), lambda i: (0, 0)),
        ],
        out_specs=pl.BlockSpec((_BLOCK_ROWS, n_classes), lambda i: (i, 0)),
        out_shape=jax.ShapeDtypeStruct((batch, n_classes), jnp.float32),
    )(x, W, b2, hist_p)
